# baseline (speedup 1.0000x reference)
import jax
import jax.numpy as jnp
from jax.experimental import pallas as pl

EMBED = 256
VOCAB = 96
CDIM = 50
WLEN = 21
KW = 5
OUT_LEN = WLEN - KW + 1
VPAD = 128
BLK = 128


def _fused_body(ids_ref, t_ref, cb_ref, wp_ref, bp_ref, wg_ref, bg_ref, out_ref):
    ids = ids_ref[...]
    iota = jax.lax.broadcasted_iota(jnp.int32, (BLK, OUT_LEN, VPAD), 2)
    ohs = [(iota == ids[:, k:k + OUT_LEN][:, :, None]).astype(jnp.float32)
           for k in range(KW)]
    x = jnp.concatenate(ohs, axis=2)
    x = x.reshape(BLK * OUT_LEN, KW * VPAD)
    conv = jax.lax.dot_general(
        x, t_ref[...], (((1,), (0,)), ((), ())),
        preferred_element_type=jnp.float32)
    conv = conv + cb_ref[...]
    h = jnp.max(jax.nn.relu(conv).reshape(BLK, OUT_LEN, EMBED), axis=1)
    proj = jax.nn.relu(
        jax.lax.dot_general(h, wp_ref[...], (((1,), (0,)), ((), ())),
                            preferred_element_type=jnp.float32) + bp_ref[...])
    gate = jax.nn.sigmoid(
        jax.lax.dot_general(h, wg_ref[...], (((1,), (0,)), ((), ())),
                            preferred_element_type=jnp.float32) + bg_ref[...])
    out_ref[...] = gate * proj + (1.0 - gate) * h


def kernel(input_ids, char_emb, conv_w, conv_b, W_proj, b_proj, W_gate, b_gate):
    sent_len, batch, wlen = input_ids.shape
    n = sent_len * batch
    ids = input_ids.reshape(n, wlen).astype(jnp.int32)

    t = jnp.einsum('vc,ock->kvo', char_emb, conv_w)
    t = jnp.pad(t, ((0, 0), (0, VPAD - VOCAB), (0, 0)))
    t = t.reshape(KW * VPAD, EMBED)

    grid = (n // BLK,)
    out = pl.pallas_call(
        _fused_body,
        grid=grid,
        in_specs=[
            pl.BlockSpec((BLK, wlen), lambda i: (i, 0)),
            pl.BlockSpec((KW * VPAD, EMBED), lambda i: (0, 0)),
            pl.BlockSpec((1, EMBED), lambda i: (0, 0)),
            pl.BlockSpec((EMBED, EMBED), lambda i: (0, 0)),
            pl.BlockSpec((1, EMBED), lambda i: (0, 0)),
            pl.BlockSpec((EMBED, EMBED), lambda i: (0, 0)),
            pl.BlockSpec((1, EMBED), lambda i: (0, 0)),
        ],
        out_specs=pl.BlockSpec((BLK, EMBED), lambda i: (i, 0)),
        out_shape=jax.ShapeDtypeStruct((n, EMBED), jnp.float32),
    )(ids, t, conv_b.reshape(1, EMBED), W_proj.T, b_proj.reshape(1, EMBED),
      W_gate.T, b_gate.reshape(1, EMBED))

    return out.reshape(sent_len, batch, EMBED)


# speedup vs baseline: 2.0346x; 2.0346x over previous
import jax
import jax.numpy as jnp
from jax.experimental import pallas as pl

EMBED = 256
VOCAB = 96
CDIM = 50
WLEN = 21
KW = 5
OUT_LEN = WLEN - KW + 1
TPAD = 24
IDPAD = TPAD + KW - 1
VPAD = 128
BLK = 128


def _fused_body(ids_ref, t_ref, cb_ref, wp_ref, bp_ref, wg_ref, bg_ref, out_ref):
    ids = ids_ref[...]
    iota = jax.lax.broadcasted_iota(jnp.int32, (BLK, TPAD, VPAD), 2)
    ohs = [(iota == ids[:, k:k + TPAD][:, :, None]).astype(jnp.float32)
           for k in range(KW)]
    x = jnp.concatenate(ohs, axis=2)
    x = x.reshape(BLK * TPAD, KW * VPAD)
    conv = jax.lax.dot_general(
        x, t_ref[...], (((1,), (0,)), ((), ())),
        preferred_element_type=jnp.float32)
    conv = jax.nn.relu(conv + cb_ref[...]).reshape(BLK, TPAD, EMBED)
    tpos = jax.lax.broadcasted_iota(jnp.int32, (BLK, TPAD, EMBED), 1)
    conv = jnp.where(tpos < OUT_LEN, conv, -jnp.inf)
    h = jnp.max(conv, axis=1)
    proj = jax.nn.relu(
        jax.lax.dot_general(h, wp_ref[...], (((1,), (0,)), ((), ())),
                            preferred_element_type=jnp.float32) + bp_ref[...])
    gate = jax.nn.sigmoid(
        jax.lax.dot_general(h, wg_ref[...], (((1,), (0,)), ((), ())),
                            preferred_element_type=jnp.float32) + bg_ref[...])
    out_ref[...] = gate * proj + (1.0 - gate) * h


def kernel(input_ids, char_emb, conv_w, conv_b, W_proj, b_proj, W_gate, b_gate):
    sent_len, batch, wlen = input_ids.shape
    n = sent_len * batch
    ids = input_ids.reshape(n, wlen).astype(jnp.int32)
    ids = jnp.pad(ids, ((0, 0), (0, IDPAD - wlen)), constant_values=-1)

    t = jnp.einsum('vc,ock->kvo', char_emb, conv_w)
    t = jnp.pad(t, ((0, 0), (0, VPAD - VOCAB), (0, 0)))
    t = t.reshape(KW * VPAD, EMBED)

    grid = (n // BLK,)
    out = pl.pallas_call(
        _fused_body,
        grid=grid,
        in_specs=[
            pl.BlockSpec((BLK, IDPAD), lambda i: (i, 0)),
            pl.BlockSpec((KW * VPAD, EMBED), lambda i: (0, 0)),
            pl.BlockSpec((1, EMBED), lambda i: (0, 0)),
            pl.BlockSpec((EMBED, EMBED), lambda i: (0, 0)),
            pl.BlockSpec((1, EMBED), lambda i: (0, 0)),
            pl.BlockSpec((EMBED, EMBED), lambda i: (0, 0)),
            pl.BlockSpec((1, EMBED), lambda i: (0, 0)),
        ],
        out_specs=pl.BlockSpec((BLK, EMBED), lambda i: (i, 0)),
        out_shape=jax.ShapeDtypeStruct((n, EMBED), jnp.float32),
    )(ids, t, conv_b.reshape(1, EMBED), W_proj.T, b_proj.reshape(1, EMBED),
      W_gate.T, b_gate.reshape(1, EMBED))

    return out.reshape(sent_len, batch, EMBED)


# speedup vs baseline: 2.1765x; 1.0698x over previous
import jax
import jax.numpy as jnp
from jax.experimental import pallas as pl

EMBED = 256
VOCAB = 96
CDIM = 50
WLEN = 21
KW = 5
OUT_LEN = WLEN - KW + 1
TPAD = 24
IDPAD = TPAD + KW - 1
VPAD = 128
BLK = 256


def _fused_body(ids_ref, t_ref, cb_ref, wp_ref, bp_ref, wg_ref, bg_ref, out_ref):
    ids = ids_ref[...]
    iota = jax.lax.broadcasted_iota(jnp.int32, (BLK, TPAD, VPAD), 2)
    ohs = [(iota == ids[:, k:k + TPAD][:, :, None]).astype(jnp.float32)
           for k in range(KW)]
    x = jnp.concatenate(ohs, axis=2)
    x = x.reshape(BLK * TPAD, KW * VPAD)
    conv = jax.lax.dot_general(
        x, t_ref[...], (((1,), (0,)), ((), ())),
        preferred_element_type=jnp.float32)
    conv = jax.nn.relu(conv + cb_ref[...]).reshape(BLK, TPAD, EMBED)
    tpos = jax.lax.broadcasted_iota(jnp.int32, (BLK, TPAD, EMBED), 1)
    conv = jnp.where(tpos < OUT_LEN, conv, -jnp.inf)
    h = jnp.max(conv, axis=1)
    proj = jax.nn.relu(
        jax.lax.dot_general(h, wp_ref[...], (((1,), (0,)), ((), ())),
                            preferred_element_type=jnp.float32) + bp_ref[...])
    gate = jax.nn.sigmoid(
        jax.lax.dot_general(h, wg_ref[...], (((1,), (0,)), ((), ())),
                            preferred_element_type=jnp.float32) + bg_ref[...])
    out_ref[...] = gate * proj + (1.0 - gate) * h


def kernel(input_ids, char_emb, conv_w, conv_b, W_proj, b_proj, W_gate, b_gate):
    sent_len, batch, wlen = input_ids.shape
    n = sent_len * batch
    ids = input_ids.reshape(n, wlen).astype(jnp.int32)
    ids = jnp.pad(ids, ((0, 0), (0, IDPAD - wlen)), constant_values=-1)

    t = jnp.einsum('vc,ock->kvo', char_emb, conv_w)
    t = jnp.pad(t, ((0, 0), (0, VPAD - VOCAB), (0, 0)))
    t = t.reshape(KW * VPAD, EMBED)

    grid = (n // BLK,)
    out = pl.pallas_call(
        _fused_body,
        grid=grid,
        in_specs=[
            pl.BlockSpec((BLK, IDPAD), lambda i: (i, 0)),
            pl.BlockSpec((KW * VPAD, EMBED), lambda i: (0, 0)),
            pl.BlockSpec((1, EMBED), lambda i: (0, 0)),
            pl.BlockSpec((EMBED, EMBED), lambda i: (0, 0)),
            pl.BlockSpec((1, EMBED), lambda i: (0, 0)),
            pl.BlockSpec((EMBED, EMBED), lambda i: (0, 0)),
            pl.BlockSpec((1, EMBED), lambda i: (0, 0)),
        ],
        out_specs=pl.BlockSpec((BLK, EMBED), lambda i: (i, 0)),
        out_shape=jax.ShapeDtypeStruct((n, EMBED), jnp.float32),
    )(ids, t, conv_b.reshape(1, EMBED), W_proj.T, b_proj.reshape(1, EMBED),
      W_gate.T, b_gate.reshape(1, EMBED))

    return out.reshape(sent_len, batch, EMBED)
